# Initial kernel scaffold; baseline (speedup 1.0000x reference)
import jax
import jax.numpy as jnp
from jax.experimental import pallas as pl


def kernel(xyz1, xyz2, points1, points2, W0, b0, g0, be0, W1, b1, g1, be1):
    raise NotImplementedError("write your pallas kernel here")


# speedup vs baseline: 16.0530x; 16.0530x over previous
import jax
import jax.numpy as jnp
from jax.experimental import pallas as pl
from jax.experimental.pallas import tpu as pltpu


def _knn_interp_conv0(xyz1_ref, xyz2t_ref, p1_ref, p2_ref, w0_ref, b0_ref,
                      x0_ref, sums_ref):
    b = pl.program_id(0)
    nb = pl.program_id(1)
    S = xyz2t_ref.shape[1]
    blkN = xyz1_ref.shape[2]

    x1 = xyz1_ref[0]
    x2t = xyz2t_ref[0]

    D = jnp.zeros((S, blkN), jnp.float32)
    for k in range(3):
        diff = x2t[:, k:k + 1] - x1[k:k + 1, :]
        D = D + diff * diff

    iota = jax.lax.broadcasted_iota(jnp.int32, (S, blkN), 0)
    INF = jnp.float32(jnp.inf)

    d1 = jnp.min(D, axis=0, keepdims=True)
    i1 = jnp.min(jnp.where(D == d1, iota, S), axis=0, keepdims=True)
    oh1 = iota == i1
    D2 = jnp.where(oh1, INF, D)
    d2 = jnp.min(D2, axis=0, keepdims=True)
    i2 = jnp.min(jnp.where(D2 == d2, iota, S), axis=0, keepdims=True)
    oh2 = iota == i2
    D3 = jnp.where(oh2, INF, D2)
    d3 = jnp.min(D3, axis=0, keepdims=True)
    i3 = jnp.min(jnp.where(D3 == d3, iota, S), axis=0, keepdims=True)
    oh3 = iota == i3

    r1 = 1.0 / (d1 + 1e-8)
    r2 = 1.0 / (d2 + 1e-8)
    r3 = 1.0 / (d3 + 1e-8)
    norm = r1 + r2 + r3
    w1 = r1 / norm
    w2 = r2 / norm
    w3 = r3 / norm

    A = (oh1.astype(jnp.float32) * w1 + oh2.astype(jnp.float32) * w2
         + oh3.astype(jnp.float32) * w3)

    interp = jax.lax.dot_general(p2_ref[0], A, (((1,), (0,)), ((), ())),
                                 preferred_element_type=jnp.float32)
    X = jnp.concatenate([p1_ref[0], interp], axis=0)
    x0 = jax.lax.dot_general(w0_ref[...], X, (((1,), (0,)), ((), ())),
                             preferred_element_type=jnp.float32)
    x0 = x0 + b0_ref[...]
    x0_ref[0] = x0

    @pl.when((b == 0) & (nb == 0))
    def _():
        sums_ref[...] = jnp.zeros_like(sums_ref)

    s = jnp.sum(x0, axis=1, keepdims=True)
    sq = jnp.sum(x0 * x0, axis=1, keepdims=True)
    sums_ref[...] += jnp.concatenate([s, sq], axis=1)


def _bn_relu_conv1(x0_ref, sums0_ref, g0_ref, be0_ref, w1_ref, b1_ref,
                   x1_ref, sums1_ref, *, count):
    b = pl.program_id(0)
    nb = pl.program_id(1)
    mean = sums0_ref[:, 0:1] / count
    var = sums0_ref[:, 1:2] / count - mean * mean
    scale = jax.lax.rsqrt(var + 1e-5) * g0_ref[...]
    x = x0_ref[0]
    xn = jnp.maximum((x - mean) * scale + be0_ref[...], 0.0)
    x1 = jax.lax.dot_general(w1_ref[...], xn, (((1,), (0,)), ((), ())),
                             preferred_element_type=jnp.float32)
    x1 = x1 + b1_ref[...]
    x1_ref[0] = x1

    @pl.when((b == 0) & (nb == 0))
    def _():
        sums1_ref[...] = jnp.zeros_like(sums1_ref)

    s = jnp.sum(x1, axis=1, keepdims=True)
    sq = jnp.sum(x1 * x1, axis=1, keepdims=True)
    sums1_ref[...] += jnp.concatenate([s, sq], axis=1)


def _bn_relu(x1_ref, sums1_ref, g1_ref, be1_ref, out_ref, *, count):
    mean = sums1_ref[:, 0:1] / count
    var = sums1_ref[:, 1:2] / count - mean * mean
    scale = jax.lax.rsqrt(var + 1e-5) * g1_ref[...]
    x = x1_ref[0]
    out_ref[0] = jnp.maximum((x - mean) * scale + be1_ref[...], 0.0)


def kernel(xyz1, xyz2, points1, points2, W0, b0, g0, be0, W1, b1, g1, be1):
    import functools

    B, _, N = xyz1.shape
    S = xyz2.shape[2]
    C1 = points1.shape[1]
    C2 = points2.shape[1]
    O0 = W0.shape[0]
    O1 = W1.shape[0]
    IN_CH = C1 + C2
    blkN = 512
    NB = N // blkN
    count = float(B * N)

    xyz2t = jnp.transpose(xyz2, (0, 2, 1))
    b0c = b0.reshape(O0, 1)
    g0c = g0.reshape(O0, 1)
    be0c = be0.reshape(O0, 1)
    b1c = b1.reshape(O1, 1)
    g1c = g1.reshape(O1, 1)
    be1c = be1.reshape(O1, 1)

    grid = (B, NB)

    x0, sums0 = pl.pallas_call(
        _knn_interp_conv0,
        grid=grid,
        in_specs=[
            pl.BlockSpec((1, 3, blkN), lambda b, n: (b, 0, n)),
            pl.BlockSpec((1, S, 3), lambda b, n: (b, 0, 0)),
            pl.BlockSpec((1, C1, blkN), lambda b, n: (b, 0, n)),
            pl.BlockSpec((1, C2, S), lambda b, n: (b, 0, 0)),
            pl.BlockSpec((O0, IN_CH), lambda b, n: (0, 0)),
            pl.BlockSpec((O0, 1), lambda b, n: (0, 0)),
        ],
        out_specs=[
            pl.BlockSpec((1, O0, blkN), lambda b, n: (b, 0, n)),
            pl.BlockSpec((O0, 2), lambda b, n: (0, 0)),
        ],
        out_shape=[
            jax.ShapeDtypeStruct((B, O0, N), jnp.float32),
            jax.ShapeDtypeStruct((O0, 2), jnp.float32),
        ],
    )(xyz1, xyz2t, points1, points2, W0, b0c)

    x1, sums1 = pl.pallas_call(
        functools.partial(_bn_relu_conv1, count=count),
        grid=grid,
        in_specs=[
            pl.BlockSpec((1, O0, blkN), lambda b, n: (b, 0, n)),
            pl.BlockSpec((O0, 2), lambda b, n: (0, 0)),
            pl.BlockSpec((O0, 1), lambda b, n: (0, 0)),
            pl.BlockSpec((O0, 1), lambda b, n: (0, 0)),
            pl.BlockSpec((O1, O0), lambda b, n: (0, 0)),
            pl.BlockSpec((O1, 1), lambda b, n: (0, 0)),
        ],
        out_specs=[
            pl.BlockSpec((1, O1, blkN), lambda b, n: (b, 0, n)),
            pl.BlockSpec((O1, 2), lambda b, n: (0, 0)),
        ],
        out_shape=[
            jax.ShapeDtypeStruct((B, O1, N), jnp.float32),
            jax.ShapeDtypeStruct((O1, 2), jnp.float32),
        ],
    )(x0, sums0, g0c, be0c, W1, b1c)

    out = pl.pallas_call(
        functools.partial(_bn_relu, count=count),
        grid=grid,
        in_specs=[
            pl.BlockSpec((1, O1, blkN), lambda b, n: (b, 0, n)),
            pl.BlockSpec((O1, 2), lambda b, n: (0, 0)),
            pl.BlockSpec((O1, 1), lambda b, n: (0, 0)),
            pl.BlockSpec((O1, 1), lambda b, n: (0, 0)),
        ],
        out_specs=pl.BlockSpec((1, O1, blkN), lambda b, n: (b, 0, n)),
        out_shape=jax.ShapeDtypeStruct((B, O1, N), jnp.float32),
    )(x1, sums1, g1c, be1c)

    return out


# speedup vs baseline: 19.0266x; 1.1852x over previous
import jax
import jax.numpy as jnp
from jax.experimental import pallas as pl
from jax.experimental.pallas import tpu as pltpu


def _knn_interp_conv0(xyz1_ref, xyz2t_ref, p1_ref, p2_ref, w0_ref, b0_ref,
                      x0_ref, sums_ref):
    b = pl.program_id(0)
    nb = pl.program_id(1)
    S = xyz2t_ref.shape[1]
    blkN = xyz1_ref.shape[2]

    x1 = xyz1_ref[0]
    x2t = xyz2t_ref[0]

    D = jnp.zeros((S, blkN), jnp.float32)
    for k in range(3):
        diff = x2t[:, k:k + 1] - x1[k:k + 1, :]
        D = D + diff * diff

    INF = jnp.float32(jnp.inf)
    d1 = jnp.min(D, axis=0, keepdims=True)
    M1 = D == d1
    D1 = jnp.where(M1, INF, D)
    d2 = jnp.min(D1, axis=0, keepdims=True)
    M2 = D1 == d2
    D2 = jnp.where(M2, INF, D1)
    d3 = jnp.min(D2, axis=0, keepdims=True)
    M3 = D2 == d3

    r1 = 1.0 / (d1 + 1e-8)
    r2 = 1.0 / (d2 + 1e-8)
    r3 = 1.0 / (d3 + 1e-8)
    norm = r1 + r2 + r3
    w1 = r1 / norm
    w2 = r2 / norm
    w3 = r3 / norm

    zero = jnp.zeros((S, blkN), jnp.float32)
    A = jnp.where(M1, w1, jnp.where(M2, w2, jnp.where(M3, w3, zero)))

    interp = jax.lax.dot_general(p2_ref[0], A, (((1,), (0,)), ((), ())),
                                 preferred_element_type=jnp.float32)
    X = jnp.concatenate([p1_ref[0], interp], axis=0)
    x0 = jax.lax.dot_general(w0_ref[...], X, (((1,), (0,)), ((), ())),
                             preferred_element_type=jnp.float32)
    x0 = x0 + b0_ref[...]
    x0_ref[0] = x0

    @pl.when((b == 0) & (nb == 0))
    def _():
        sums_ref[...] = jnp.zeros_like(sums_ref)

    s = jnp.sum(x0, axis=1, keepdims=True)
    sq = jnp.sum(x0 * x0, axis=1, keepdims=True)
    sums_ref[...] += jnp.concatenate([s, sq], axis=1)


def _bn_relu_conv1(x0_ref, sums0_ref, g0_ref, be0_ref, w1_ref, b1_ref,
                   x1_ref, sums1_ref, *, count):
    b = pl.program_id(0)
    nb = pl.program_id(1)
    mean = sums0_ref[:, 0:1] / count
    var = sums0_ref[:, 1:2] / count - mean * mean
    scale = jax.lax.rsqrt(var + 1e-5) * g0_ref[...]
    x = x0_ref[0]
    xn = jnp.maximum((x - mean) * scale + be0_ref[...], 0.0)
    x1 = jax.lax.dot_general(w1_ref[...], xn, (((1,), (0,)), ((), ())),
                             preferred_element_type=jnp.float32)
    x1 = x1 + b1_ref[...]
    x1_ref[0] = x1

    @pl.when((b == 0) & (nb == 0))
    def _():
        sums1_ref[...] = jnp.zeros_like(sums1_ref)

    s = jnp.sum(x1, axis=1, keepdims=True)
    sq = jnp.sum(x1 * x1, axis=1, keepdims=True)
    sums1_ref[...] += jnp.concatenate([s, sq], axis=1)


def _bn_relu(x1_ref, sums1_ref, g1_ref, be1_ref, out_ref, *, count):
    mean = sums1_ref[:, 0:1] / count
    var = sums1_ref[:, 1:2] / count - mean * mean
    scale = jax.lax.rsqrt(var + 1e-5) * g1_ref[...]
    x = x1_ref[0]
    out_ref[0] = jnp.maximum((x - mean) * scale + be1_ref[...], 0.0)


def kernel(xyz1, xyz2, points1, points2, W0, b0, g0, be0, W1, b1, g1, be1):
    import functools

    B, _, N = xyz1.shape
    S = xyz2.shape[2]
    C1 = points1.shape[1]
    C2 = points2.shape[1]
    O0 = W0.shape[0]
    O1 = W1.shape[0]
    IN_CH = C1 + C2
    blkN = 512
    NB = N // blkN
    count = float(B * N)

    xyz2t = jnp.transpose(xyz2, (0, 2, 1))
    b0c = b0.reshape(O0, 1)
    g0c = g0.reshape(O0, 1)
    be0c = be0.reshape(O0, 1)
    b1c = b1.reshape(O1, 1)
    g1c = g1.reshape(O1, 1)
    be1c = be1.reshape(O1, 1)

    grid = (B, NB)

    x0, sums0 = pl.pallas_call(
        _knn_interp_conv0,
        grid=grid,
        in_specs=[
            pl.BlockSpec((1, 3, blkN), lambda b, n: (b, 0, n)),
            pl.BlockSpec((1, S, 3), lambda b, n: (b, 0, 0)),
            pl.BlockSpec((1, C1, blkN), lambda b, n: (b, 0, n)),
            pl.BlockSpec((1, C2, S), lambda b, n: (b, 0, 0)),
            pl.BlockSpec((O0, IN_CH), lambda b, n: (0, 0)),
            pl.BlockSpec((O0, 1), lambda b, n: (0, 0)),
        ],
        out_specs=[
            pl.BlockSpec((1, O0, blkN), lambda b, n: (b, 0, n)),
            pl.BlockSpec((O0, 2), lambda b, n: (0, 0)),
        ],
        out_shape=[
            jax.ShapeDtypeStruct((B, O0, N), jnp.float32),
            jax.ShapeDtypeStruct((O0, 2), jnp.float32),
        ],
    )(xyz1, xyz2t, points1, points2, W0, b0c)

    x1, sums1 = pl.pallas_call(
        functools.partial(_bn_relu_conv1, count=count),
        grid=grid,
        in_specs=[
            pl.BlockSpec((1, O0, blkN), lambda b, n: (b, 0, n)),
            pl.BlockSpec((O0, 2), lambda b, n: (0, 0)),
            pl.BlockSpec((O0, 1), lambda b, n: (0, 0)),
            pl.BlockSpec((O0, 1), lambda b, n: (0, 0)),
            pl.BlockSpec((O1, O0), lambda b, n: (0, 0)),
            pl.BlockSpec((O1, 1), lambda b, n: (0, 0)),
        ],
        out_specs=[
            pl.BlockSpec((1, O1, blkN), lambda b, n: (b, 0, n)),
            pl.BlockSpec((O1, 2), lambda b, n: (0, 0)),
        ],
        out_shape=[
            jax.ShapeDtypeStruct((B, O1, N), jnp.float32),
            jax.ShapeDtypeStruct((O1, 2), jnp.float32),
        ],
    )(x0, sums0, g0c, be0c, W1, b1c)

    out = pl.pallas_call(
        functools.partial(_bn_relu, count=count),
        grid=grid,
        in_specs=[
            pl.BlockSpec((1, O1, blkN), lambda b, n: (b, 0, n)),
            pl.BlockSpec((O1, 2), lambda b, n: (0, 0)),
            pl.BlockSpec((O1, 1), lambda b, n: (0, 0)),
            pl.BlockSpec((O1, 1), lambda b, n: (0, 0)),
        ],
        out_specs=pl.BlockSpec((1, O1, blkN), lambda b, n: (b, 0, n)),
        out_shape=jax.ShapeDtypeStruct((B, O1, N), jnp.float32),
    )(x1, sums1, g1c, be1c)

    return out


# speedup vs baseline: 21.7546x; 1.1434x over previous
import functools

import jax
import jax.numpy as jnp
from jax.experimental import pallas as pl
from jax.experimental.pallas import tpu as pltpu


def _fused(xyz1_ref, xyz2t_ref, p1_ref, p2_ref, w0_ref, b0_ref, g0_ref,
           be0_ref, w1_ref, b1_ref, g1_ref, be1_ref, out_ref,
           x0s, x1s, sums0, sums1, *, count, blkN):
    p = pl.program_id(0)
    b = pl.program_id(1)
    nb = pl.program_id(2)
    NB = pl.num_programs(2)
    S = xyz2t_ref.shape[1]
    col = pl.ds((b * NB + nb) * blkN, blkN)

    @pl.when((p == 0) & (b == 0) & (nb == 0))
    def _():
        sums0[...] = jnp.zeros_like(sums0)
        sums1[...] = jnp.zeros_like(sums1)

    @pl.when(p == 0)
    def _phase0():
        x1 = xyz1_ref[0]
        x2t = xyz2t_ref[0]
        D = jnp.zeros((S, blkN), jnp.float32)
        for k in range(3):
            diff = x2t[:, k:k + 1] - x1[k:k + 1, :]
            D = D + diff * diff

        INF = jnp.float32(jnp.inf)
        d1 = jnp.min(D, axis=0, keepdims=True)
        M1 = D == d1
        D1 = jnp.where(M1, INF, D)
        d2 = jnp.min(D1, axis=0, keepdims=True)
        M2 = D1 == d2
        D2 = jnp.where(M2, INF, D1)
        d3 = jnp.min(D2, axis=0, keepdims=True)
        M3 = D2 == d3

        r1 = 1.0 / (d1 + 1e-8)
        r2 = 1.0 / (d2 + 1e-8)
        r3 = 1.0 / (d3 + 1e-8)
        norm = r1 + r2 + r3
        w1 = r1 / norm
        w2 = r2 / norm
        w3 = r3 / norm

        zero = jnp.zeros((S, blkN), jnp.float32)
        A = jnp.where(M1, w1, jnp.where(M2, w2, jnp.where(M3, w3, zero)))

        interp = jax.lax.dot_general(p2_ref[0], A, (((1,), (0,)), ((), ())),
                                     preferred_element_type=jnp.float32)
        X = jnp.concatenate([p1_ref[0], interp], axis=0)
        x0 = jax.lax.dot_general(w0_ref[...], X, (((1,), (0,)), ((), ())),
                                 preferred_element_type=jnp.float32)
        x0 = x0 + b0_ref[...]
        x0s[:, col] = x0
        s = jnp.sum(x0, axis=1, keepdims=True)
        sq = jnp.sum(x0 * x0, axis=1, keepdims=True)
        sums0[...] += jnp.concatenate([s, sq], axis=1)

    @pl.when(p == 1)
    def _phase1():
        mean = sums0[:, 0:1] / count
        var = sums0[:, 1:2] / count - mean * mean
        scale = jax.lax.rsqrt(var + 1e-5) * g0_ref[...]
        xn = jnp.maximum((x0s[:, col] - mean) * scale + be0_ref[...], 0.0)
        x1 = jax.lax.dot_general(w1_ref[...], xn, (((1,), (0,)), ((), ())),
                                 preferred_element_type=jnp.float32)
        x1 = x1 + b1_ref[...]
        x1s[:, col] = x1
        s = jnp.sum(x1, axis=1, keepdims=True)
        sq = jnp.sum(x1 * x1, axis=1, keepdims=True)
        sums1[...] += jnp.concatenate([s, sq], axis=1)

    @pl.when(p == 2)
    def _phase2():
        mean = sums1[:, 0:1] / count
        var = sums1[:, 1:2] / count - mean * mean
        scale = jax.lax.rsqrt(var + 1e-5) * g1_ref[...]
        out_ref[0] = jnp.maximum((x1s[:, col] - mean) * scale + be1_ref[...],
                                 0.0)


def kernel(xyz1, xyz2, points1, points2, W0, b0, g0, be0, W1, b1, g1, be1):
    B, _, N = xyz1.shape
    S = xyz2.shape[2]
    C1 = points1.shape[1]
    C2 = points2.shape[1]
    O0 = W0.shape[0]
    O1 = W1.shape[0]
    IN_CH = C1 + C2
    blkN = 512
    NB = N // blkN
    count = float(B * N)

    xyz2t = jnp.transpose(xyz2, (0, 2, 1))
    b0c = b0.reshape(O0, 1)
    g0c = g0.reshape(O0, 1)
    be0c = be0.reshape(O0, 1)
    b1c = b1.reshape(O1, 1)
    g1c = g1.reshape(O1, 1)
    be1c = be1.reshape(O1, 1)

    def p0_map(p, b, n):
        z = (p == 0).astype(jnp.int32)
        return (b * z, 0, n * z)

    out = pl.pallas_call(
        functools.partial(_fused, count=count, blkN=blkN),
        grid=(3, B, NB),
        in_specs=[
            pl.BlockSpec((1, 3, blkN), p0_map),
            pl.BlockSpec((1, S, 3), lambda p, b, n: (b * (p == 0), 0, 0)),
            pl.BlockSpec((1, C1, blkN), p0_map),
            pl.BlockSpec((1, C2, S), lambda p, b, n: (b * (p == 0), 0, 0)),
            pl.BlockSpec((O0, IN_CH), lambda p, b, n: (0, 0)),
            pl.BlockSpec((O0, 1), lambda p, b, n: (0, 0)),
            pl.BlockSpec((O0, 1), lambda p, b, n: (0, 0)),
            pl.BlockSpec((O0, 1), lambda p, b, n: (0, 0)),
            pl.BlockSpec((O1, O0), lambda p, b, n: (0, 0)),
            pl.BlockSpec((O1, 1), lambda p, b, n: (0, 0)),
            pl.BlockSpec((O1, 1), lambda p, b, n: (0, 0)),
            pl.BlockSpec((O1, 1), lambda p, b, n: (0, 0)),
        ],
        out_specs=pl.BlockSpec((1, O1, blkN),
                               lambda p, b, n: (b * (p == 2), 0,
                                                n * (p == 2))),
        out_shape=jax.ShapeDtypeStruct((B, O1, N), jnp.float32),
        scratch_shapes=[
            pltpu.VMEM((O0, B * N), jnp.float32),
            pltpu.VMEM((O1, B * N), jnp.float32),
            pltpu.VMEM((O0, 2), jnp.float32),
            pltpu.VMEM((O1, 2), jnp.float32),
        ],
    )(xyz1, xyz2t, points1, points2, W0, b0c, g0c, be0c, W1, b1c, g1c, be1c)

    return out


# speedup vs baseline: 29.4529x; 1.3539x over previous
import functools

import jax
import jax.numpy as jnp
from jax.experimental import pallas as pl
from jax.experimental.pallas import tpu as pltpu


def _fused(xyz1_ref, xyz2t_ref, p1_ref, p2_ref, w0_ref, b0_ref, g0_ref,
           be0_ref, w1_ref, b1_ref, g1_ref, be1_ref, out_ref,
           x0s, x1s, sums0, sums1, *, count, blkN):
    p = pl.program_id(0)
    b = pl.program_id(1)
    nb = pl.program_id(2)
    NB = pl.num_programs(2)
    S = xyz2t_ref.shape[1]
    col = pl.ds((b * NB + nb) * blkN, blkN)

    @pl.when((p == 0) & (b == 0) & (nb == 0))
    def _():
        sums0[...] = jnp.zeros_like(sums0)
        sums1[...] = jnp.zeros_like(sums1)

    @pl.when(p == 0)
    def _phase0():
        x1 = xyz1_ref[0]
        x2t = xyz2t_ref[0]
        D = jnp.zeros((S, blkN), jnp.float32)
        for k in range(3):
            diff = x2t[:, k:k + 1] - x1[k:k + 1, :]
            D = D + diff * diff

        INF = jnp.float32(jnp.inf)
        d1 = jnp.min(D, axis=0, keepdims=True)
        M1 = D == d1
        D1 = jnp.where(M1, INF, D)
        d2 = jnp.min(D1, axis=0, keepdims=True)
        M2 = D1 == d2
        D2 = jnp.where(M2, INF, D1)
        d3 = jnp.min(D2, axis=0, keepdims=True)
        M3 = D2 == d3

        r1 = 1.0 / (d1 + 1e-8)
        r2 = 1.0 / (d2 + 1e-8)
        r3 = 1.0 / (d3 + 1e-8)
        norm = r1 + r2 + r3
        w1 = r1 / norm
        w2 = r2 / norm
        w3 = r3 / norm

        zero = jnp.zeros((S, blkN), jnp.float32)
        A = jnp.where(M1, w1, jnp.where(M2, w2, jnp.where(M3, w3, zero)))

        interp = jax.lax.dot_general(p2_ref[0], A, (((1,), (0,)), ((), ())),
                                     preferred_element_type=jnp.float32)
        X = jnp.concatenate([p1_ref[0], interp], axis=0)
        x0 = jax.lax.dot_general(w0_ref[...], X, (((1,), (0,)), ((), ())),
                                 preferred_element_type=jnp.float32)
        x0 = x0 + b0_ref[...]
        x0s[:, col] = x0
        s = jnp.sum(x0, axis=1, keepdims=True)
        sq = jnp.sum(x0 * x0, axis=1, keepdims=True)
        sums0[...] += jnp.concatenate([s, sq], axis=1)

    @pl.when(p == 1)
    def _phase1():
        mean = sums0[:, 0:1] / count
        var = sums0[:, 1:2] / count - mean * mean
        scale = jax.lax.rsqrt(var + 1e-5) * g0_ref[...]
        xn = jnp.maximum((x0s[:, col] - mean) * scale + be0_ref[...], 0.0)
        x1 = jax.lax.dot_general(w1_ref[...], xn, (((1,), (0,)), ((), ())),
                                 preferred_element_type=jnp.float32)
        x1 = x1 + b1_ref[...]
        x1s[:, col] = x1
        s = jnp.sum(x1, axis=1, keepdims=True)
        sq = jnp.sum(x1 * x1, axis=1, keepdims=True)
        sums1[...] += jnp.concatenate([s, sq], axis=1)

    @pl.when(p == 2)
    def _phase2():
        mean = sums1[:, 0:1] / count
        var = sums1[:, 1:2] / count - mean * mean
        scale = jax.lax.rsqrt(var + 1e-5) * g1_ref[...]
        out_ref[0] = jnp.maximum((x1s[:, col] - mean) * scale + be1_ref[...],
                                 0.0)


def kernel(xyz1, xyz2, points1, points2, W0, b0, g0, be0, W1, b1, g1, be1):
    B, _, N = xyz1.shape
    S = xyz2.shape[2]
    C1 = points1.shape[1]
    C2 = points2.shape[1]
    O0 = W0.shape[0]
    O1 = W1.shape[0]
    IN_CH = C1 + C2
    blkN = 1024
    NB = N // blkN
    count = float(B * N)

    xyz2t = jnp.transpose(xyz2, (0, 2, 1))
    b0c = b0.reshape(O0, 1)
    g0c = g0.reshape(O0, 1)
    be0c = be0.reshape(O0, 1)
    b1c = b1.reshape(O1, 1)
    g1c = g1.reshape(O1, 1)
    be1c = be1.reshape(O1, 1)

    def p0_map(p, b, n):
        z = (p == 0).astype(jnp.int32)
        return (b * z, 0, n * z)

    out = pl.pallas_call(
        functools.partial(_fused, count=count, blkN=blkN),
        grid=(3, B, NB),
        in_specs=[
            pl.BlockSpec((1, 3, blkN), p0_map),
            pl.BlockSpec((1, S, 3), lambda p, b, n: (b * (p == 0), 0, 0)),
            pl.BlockSpec((1, C1, blkN), p0_map),
            pl.BlockSpec((1, C2, S), lambda p, b, n: (b * (p == 0), 0, 0)),
            pl.BlockSpec((O0, IN_CH), lambda p, b, n: (0, 0)),
            pl.BlockSpec((O0, 1), lambda p, b, n: (0, 0)),
            pl.BlockSpec((O0, 1), lambda p, b, n: (0, 0)),
            pl.BlockSpec((O0, 1), lambda p, b, n: (0, 0)),
            pl.BlockSpec((O1, O0), lambda p, b, n: (0, 0)),
            pl.BlockSpec((O1, 1), lambda p, b, n: (0, 0)),
            pl.BlockSpec((O1, 1), lambda p, b, n: (0, 0)),
            pl.BlockSpec((O1, 1), lambda p, b, n: (0, 0)),
        ],
        out_specs=pl.BlockSpec((1, O1, blkN),
                               lambda p, b, n: (b * (p == 2), 0,
                                                n * (p == 2))),
        out_shape=jax.ShapeDtypeStruct((B, O1, N), jnp.float32),
        scratch_shapes=[
            pltpu.VMEM((O0, B * N), jnp.float32),
            pltpu.VMEM((O1, B * N), jnp.float32),
            pltpu.VMEM((O0, 2), jnp.float32),
            pltpu.VMEM((O1, 2), jnp.float32),
        ],
    )(xyz1, xyz2t, points1, points2, W0, b0c, g0c, be0c, W1, b1c, g1c, be1c)

    return out


# speedup vs baseline: 32.0549x; 1.0883x over previous
import functools

import jax
import jax.numpy as jnp
from jax.experimental import pallas as pl
from jax.experimental.pallas import tpu as pltpu


def _fused(xyz1_ref, xyz2t_ref, p1_ref, p2_ref, w0_ref, b0_ref, g0_ref,
           be0_ref, w1_ref, b1_ref, g1_ref, be1_ref, out_ref,
           x0s, x1s, sums0, sums1, *, count, blkN):
    p = pl.program_id(0)
    b = pl.program_id(1)
    nb = pl.program_id(2)
    NB = pl.num_programs(2)
    S = xyz2t_ref.shape[1]
    col = pl.ds((b * NB + nb) * blkN, blkN)

    @pl.when((p == 0) & (b == 0) & (nb == 0))
    def _():
        sums0[...] = jnp.zeros_like(sums0)
        sums1[...] = jnp.zeros_like(sums1)

    @pl.when(p == 0)
    def _phase0():
        x1 = xyz1_ref[0]
        x2t = xyz2t_ref[0]
        D = jnp.zeros((S, blkN), jnp.float32)
        for k in range(3):
            diff = x2t[:, k:k + 1] - x1[k:k + 1, :]
            D = D + diff * diff

        INF = jnp.float32(jnp.inf)
        d1 = jnp.min(D, axis=0, keepdims=True)
        M1 = D == d1
        D1 = jnp.where(M1, INF, D)
        d2 = jnp.min(D1, axis=0, keepdims=True)
        M2 = D1 == d2
        D2 = jnp.where(M2, INF, D1)
        d3 = jnp.min(D2, axis=0, keepdims=True)
        M3 = D2 == d3

        r1 = 1.0 / (d1 + 1e-8)
        r2 = 1.0 / (d2 + 1e-8)
        r3 = 1.0 / (d3 + 1e-8)
        norm = r1 + r2 + r3
        w1 = r1 / norm
        w2 = r2 / norm
        w3 = r3 / norm

        zero = jnp.zeros((S, blkN), jnp.float32)
        A = jnp.where(M1, w1, jnp.where(M2, w2, jnp.where(M3, w3, zero)))

        interp = jax.lax.dot_general(p2_ref[0], A, (((1,), (0,)), ((), ())),
                                     preferred_element_type=jnp.float32)
        X = jnp.concatenate([p1_ref[0], interp], axis=0)
        x0 = jax.lax.dot_general(w0_ref[...], X, (((1,), (0,)), ((), ())),
                                 preferred_element_type=jnp.float32)
        x0 = x0 + b0_ref[...]
        x0s[:, col] = x0
        s = jnp.sum(x0, axis=1, keepdims=True)
        sq = jnp.sum(x0 * x0, axis=1, keepdims=True)
        sums0[...] += jnp.concatenate([s, sq], axis=1)

    @pl.when(p == 1)
    def _phase1():
        mean = sums0[:, 0:1] / count
        var = sums0[:, 1:2] / count - mean * mean
        scale = jax.lax.rsqrt(var + 1e-5) * g0_ref[...]
        xn = jnp.maximum((x0s[:, col] - mean) * scale + be0_ref[...], 0.0)
        x1 = jax.lax.dot_general(w1_ref[...], xn, (((1,), (0,)), ((), ())),
                                 preferred_element_type=jnp.float32)
        x1 = x1 + b1_ref[...]
        x1s[:, col] = x1
        s = jnp.sum(x1, axis=1, keepdims=True)
        sq = jnp.sum(x1 * x1, axis=1, keepdims=True)
        sums1[...] += jnp.concatenate([s, sq], axis=1)

    @pl.when(p == 2)
    def _phase2():
        mean = sums1[:, 0:1] / count
        var = sums1[:, 1:2] / count - mean * mean
        scale = jax.lax.rsqrt(var + 1e-5) * g1_ref[...]
        out_ref[0] = jnp.maximum((x1s[:, col] - mean) * scale + be1_ref[...],
                                 0.0)


def kernel(xyz1, xyz2, points1, points2, W0, b0, g0, be0, W1, b1, g1, be1):
    B, _, N = xyz1.shape
    S = xyz2.shape[2]
    C1 = points1.shape[1]
    C2 = points2.shape[1]
    O0 = W0.shape[0]
    O1 = W1.shape[0]
    IN_CH = C1 + C2
    blkN = 2048
    NB = N // blkN
    count = float(B * N)

    xyz2t = jnp.transpose(xyz2, (0, 2, 1))
    b0c = b0.reshape(O0, 1)
    g0c = g0.reshape(O0, 1)
    be0c = be0.reshape(O0, 1)
    b1c = b1.reshape(O1, 1)
    g1c = g1.reshape(O1, 1)
    be1c = be1.reshape(O1, 1)

    def p0_map(p, b, n):
        z = (p == 0).astype(jnp.int32)
        return (b * z, 0, n * z)

    out = pl.pallas_call(
        functools.partial(_fused, count=count, blkN=blkN),
        grid=(3, B, NB),
        in_specs=[
            pl.BlockSpec((1, 3, blkN), p0_map),
            pl.BlockSpec((1, S, 3), lambda p, b, n: (b * (p == 0), 0, 0)),
            pl.BlockSpec((1, C1, blkN), p0_map),
            pl.BlockSpec((1, C2, S), lambda p, b, n: (b * (p == 0), 0, 0)),
            pl.BlockSpec((O0, IN_CH), lambda p, b, n: (0, 0)),
            pl.BlockSpec((O0, 1), lambda p, b, n: (0, 0)),
            pl.BlockSpec((O0, 1), lambda p, b, n: (0, 0)),
            pl.BlockSpec((O0, 1), lambda p, b, n: (0, 0)),
            pl.BlockSpec((O1, O0), lambda p, b, n: (0, 0)),
            pl.BlockSpec((O1, 1), lambda p, b, n: (0, 0)),
            pl.BlockSpec((O1, 1), lambda p, b, n: (0, 0)),
            pl.BlockSpec((O1, 1), lambda p, b, n: (0, 0)),
        ],
        out_specs=pl.BlockSpec((1, O1, blkN),
                               lambda p, b, n: (b * (p == 2), 0,
                                                n * (p == 2))),
        out_shape=jax.ShapeDtypeStruct((B, O1, N), jnp.float32),
        scratch_shapes=[
            pltpu.VMEM((O0, B * N), jnp.float32),
            pltpu.VMEM((O1, B * N), jnp.float32),
            pltpu.VMEM((O0, 2), jnp.float32),
            pltpu.VMEM((O1, 2), jnp.float32),
        ],
    )(xyz1, xyz2t, points1, points2, W0, b0c, g0c, be0c, W1, b1c, g1c, be1c)

    return out
